# Initial kernel scaffold; baseline (speedup 1.0000x reference)
import jax
import jax.numpy as jnp
from jax.experimental import pallas as pl


def kernel(x, edge_index, W_edge, b_edge, W_node, b_node, W_cat1, b_cat1, W_cat2, b_cat2, W_final, b_final):
    raise NotImplementedError("write your pallas kernel here")


# speedup vs baseline: 5.6200x; 5.6200x over previous
import functools

import jax
import jax.numpy as jnp
from jax import lax
from jax.experimental import pallas as pl
from jax.experimental.pallas import tpu as pltpu
import jax.experimental.pallas.tpu_sc as plsc

N = 10000
E = 320000
D = 128
H = 128
OUT = 128

NC = 2
NS = 16
NW = NC * NS
EPW = E // NW
C = 80
NCHUNK = EPW // C
RB = 80
NRB = N // RB


def _sc_segment_sum(W_edge, src, dst):
    mesh = plsc.VectorSubcoreMesh(
        core_axis_name="c", subcore_axis_name="s",
        num_cores=NC, num_subcores=NS)

    @functools.partial(
        pl.kernel,
        mesh=mesh,
        out_type=jax.ShapeDtypeStruct((NC, N, H), jnp.float32),
        scratch_types=[
            pltpu.VMEM((C,), jnp.int32),
            pltpu.VMEM((C,), jnp.int32),
            pltpu.VMEM((C, H), jnp.float32),
            pltpu.VMEM_SHARED((N, H), jnp.float32),
            pltpu.SemaphoreType.DMA,
        ],
    )
    def k(w_hbm, src_hbm, dst_hbm, out_hbm, src_v, dst_v, rows_v, acc, sem):
        cid = lax.axis_index("c")
        sid = lax.axis_index("s")
        wid = sid * NC + cid

        def zero_row(r, carry):
            for g in range(H // 16):
                rows_v[r, pl.ds(g * 16, 16)] = jnp.zeros((16,), jnp.float32)
            return carry
        lax.fori_loop(0, C, zero_row, 0)

        for j in range((NRB + NS - 1) // NS):
            g = sid + j * NS
            @pl.when(g < NRB)
            def _():
                pltpu.sync_copy(rows_v, acc.at[pl.ds(g * RB, RB)])
        plsc.subcore_barrier()

        def chunk(i, carry):
            base = wid * EPW + i * C
            pltpu.sync_copy(src_hbm.at[pl.ds(base, C)], src_v)
            pltpu.sync_copy(dst_hbm.at[pl.ds(base, C)], dst_v)
            pltpu.async_copy(w_hbm.at[src_v], rows_v, sem).wait()
            pltpu.sync_copy(rows_v, acc.at[dst_v], add=True)
            return carry
        lax.fori_loop(0, NCHUNK, chunk, 0)
        plsc.subcore_barrier()

        for j in range((NRB + NS - 1) // NS):
            g = sid + j * NS
            @pl.when(g < NRB)
            def _():
                pltpu.sync_copy(acc.at[pl.ds(g * RB, RB)],
                                out_hbm.at[cid, pl.ds(g * RB, RB)])

    return k(W_edge, src, dst)


BN = 2000


def _tc_tail_kernel(s_ref, x_ref, wc1_ref, bc1_ref, wn_ref, bn_ref,
                    wc2_ref, bc2_ref, wf_ref, bf_ref, be_ref, out_ref):
    s = s_ref[0] + s_ref[1] + be_ref[...]
    t = s + jnp.dot(s, wc1_ref[...], preferred_element_type=jnp.float32) \
        + bc1_ref[...]
    h = jnp.dot(x_ref[...], wn_ref[...], preferred_element_type=jnp.float32) \
        + bn_ref[...]
    t = t + h + jnp.dot(h, wc2_ref[...], preferred_element_type=jnp.float32) \
        + bc2_ref[...]
    t = jnp.maximum(t, 0.0)
    out_ref[...] = jnp.dot(t, wf_ref[...],
                           preferred_element_type=jnp.float32) + bf_ref[...]


def _tc_tail(S2, x, W_cat1, b_cat1, W_node, b_node, W_cat2, b_cat2,
             W_final, b_final, b_edge):
    full = lambda shape: pl.BlockSpec(shape, lambda i: (0, 0))
    return pl.pallas_call(
        _tc_tail_kernel,
        grid=(N // BN,),
        in_specs=[
            pl.BlockSpec((NC, BN, H), lambda i: (0, i, 0)),
            pl.BlockSpec((BN, D), lambda i: (i, 0)),
            full((H, H)), full((1, H)),
            full((D, H)), full((1, H)),
            full((H, H)), full((1, H)),
            full((H, OUT)), full((1, OUT)),
            full((1, H)),
        ],
        out_specs=pl.BlockSpec((BN, OUT), lambda i: (i, 0)),
        out_shape=jax.ShapeDtypeStruct((N, OUT), jnp.float32),
    )(S2, x, W_cat1, b_cat1, W_node, b_node, W_cat2, b_cat2,
      W_final, b_final, b_edge)


def kernel(x, edge_index, W_edge, b_edge, W_node, b_node,
           W_cat1, b_cat1, W_cat2, b_cat2, W_final, b_final):
    src = edge_index[0]
    dst = edge_index[1]
    S2 = _sc_segment_sum(W_edge, src, dst)
    return _tc_tail(S2, x,
                    W_cat1, b_cat1.reshape(1, H),
                    W_node, b_node.reshape(1, H),
                    W_cat2, b_cat2.reshape(1, H),
                    W_final, b_final.reshape(1, OUT),
                    b_edge.reshape(1, H))


# speedup vs baseline: 7.7327x; 1.3759x over previous
import functools

import jax
import jax.numpy as jnp
from jax import lax
from jax.experimental import pallas as pl
from jax.experimental.pallas import tpu as pltpu
import jax.experimental.pallas.tpu_sc as plsc

N = 10000
E = 320000
D = 128
H = 128
OUT = 128

NC = 2
NS = 16
NW = NC * NS
EPW = E // NW
C = 40
NCHUNK = EPW // C
RB = 40
NRB = N // RB


def _sc_segment_sum(W_edge, src, dst):
    mesh = plsc.VectorSubcoreMesh(
        core_axis_name="c", subcore_axis_name="s",
        num_cores=NC, num_subcores=NS)

    @functools.partial(
        pl.kernel,
        mesh=mesh,
        out_type=jax.ShapeDtypeStruct((NC, N, H), jnp.float32),
        scratch_types=[
            pltpu.VMEM((2, C), jnp.int32),
            pltpu.VMEM((2, C), jnp.int32),
            pltpu.VMEM((C, H), jnp.float32),
            pltpu.VMEM((C, H), jnp.float32),
            pltpu.VMEM_SHARED((N, H), jnp.float32),
            pltpu.SemaphoreType.DMA((2,)),
            pltpu.SemaphoreType.DMA((2,)),
            pltpu.SemaphoreType.DMA((2,)),
        ],
    )
    def k(w_hbm, src_hbm, dst_hbm, out_hbm,
          src_v, dst_v, rows0, rows1, acc, semS, semD, semG):
        cid = lax.axis_index("c")
        sid = lax.axis_index("s")
        wid = sid * NC + cid
        rows = (rows0, rows1)

        def load_idx(i, b):
            base = wid * EPW + i * C
            pltpu.async_copy(src_hbm.at[pl.ds(base, C)], src_v.at[b],
                             semS.at[b])
            pltpu.async_copy(dst_hbm.at[pl.ds(base, C)], dst_v.at[b],
                             semD.at[b])

        def wait_idx(i, b):
            base = wid * EPW + i * C
            pltpu.make_async_copy(src_hbm.at[pl.ds(base, C)], src_v.at[b],
                                  semS.at[b]).wait()
            pltpu.make_async_copy(dst_hbm.at[pl.ds(base, C)], dst_v.at[b],
                                  semD.at[b]).wait()

        def gather(b):
            pltpu.async_copy(w_hbm.at[src_v.at[b]], rows[b], semG.at[b])

        def wait_gather(b):
            pltpu.make_async_copy(w_hbm.at[src_v.at[b]], rows[b],
                                  semG.at[b]).wait()

        def scat(b):
            pltpu.sync_copy(rows[b], acc.at[dst_v.at[b]], add=True)

        load_idx(0, 0)
        load_idx(1, 1)

        def zero_row(r, carry):
            for g in range(H // 16):
                rows0[r, pl.ds(g * 16, 16)] = jnp.zeros((16,), jnp.float32)
            return carry
        lax.fori_loop(0, RB, zero_row, 0)
        for j in range((NRB + NS - 1) // NS):
            g = sid + j * NS
            @pl.when(g < NRB)
            def _():
                pltpu.sync_copy(rows0.at[pl.ds(0, RB)],
                                acc.at[pl.ds(g * RB, RB)])
        wait_idx(0, 0)
        gather(0)
        plsc.subcore_barrier()

        def half(i, b):
            nb = 1 - b
            wait_idx(i + 1, nb)
            gather(nb)
            wait_gather(b)
            @pl.when(i + 2 < NCHUNK)
            def _():
                pltpu.async_copy(
                    src_hbm.at[pl.ds(wid * EPW + (i + 2) * C, C)],
                    src_v.at[b], semS.at[b])
            scat(b)
            @pl.when(i + 2 < NCHUNK)
            def _():
                pltpu.async_copy(
                    dst_hbm.at[pl.ds(wid * EPW + (i + 2) * C, C)],
                    dst_v.at[b], semD.at[b])

        def body(p, carry):
            half(2 * p, 0)
            half(2 * p + 1, 1)
            return carry
        lax.fori_loop(0, (NCHUNK - 2) // 2, body, 0)
        i = NCHUNK - 2
        wait_idx(i + 1, 1)
        gather(1)
        wait_gather(0)
        scat(0)
        wait_gather(1)
        scat(1)
        plsc.subcore_barrier()

        for j in range((NRB + NS - 1) // NS):
            g = sid + j * NS
            @pl.when(g < NRB)
            def _():
                pltpu.sync_copy(acc.at[pl.ds(g * RB, RB)],
                                out_hbm.at[cid, pl.ds(g * RB, RB)])

    return k(W_edge, src, dst)


BN = 2000


def _tc_tail_kernel(s_ref, x_ref, wc1_ref, bc1_ref, wn_ref, bn_ref,
                    wc2_ref, bc2_ref, wf_ref, bf_ref, be_ref, out_ref):
    s = s_ref[0] + s_ref[1] + be_ref[...]
    t = s + jnp.dot(s, wc1_ref[...], preferred_element_type=jnp.float32) \
        + bc1_ref[...]
    h = jnp.dot(x_ref[...], wn_ref[...], preferred_element_type=jnp.float32) \
        + bn_ref[...]
    t = t + h + jnp.dot(h, wc2_ref[...], preferred_element_type=jnp.float32) \
        + bc2_ref[...]
    t = jnp.maximum(t, 0.0)
    out_ref[...] = jnp.dot(t, wf_ref[...],
                           preferred_element_type=jnp.float32) + bf_ref[...]


def _tc_tail(S2, x, W_cat1, b_cat1, W_node, b_node, W_cat2, b_cat2,
             W_final, b_final, b_edge):
    full = lambda shape: pl.BlockSpec(shape, lambda i: (0, 0))
    return pl.pallas_call(
        _tc_tail_kernel,
        grid=(N // BN,),
        in_specs=[
            pl.BlockSpec((NC, BN, H), lambda i: (0, i, 0)),
            pl.BlockSpec((BN, D), lambda i: (i, 0)),
            full((H, H)), full((1, H)),
            full((D, H)), full((1, H)),
            full((H, H)), full((1, H)),
            full((H, OUT)), full((1, OUT)),
            full((1, H)),
        ],
        out_specs=pl.BlockSpec((BN, OUT), lambda i: (i, 0)),
        out_shape=jax.ShapeDtypeStruct((N, OUT), jnp.float32),
    )(S2, x, W_cat1, b_cat1, W_node, b_node, W_cat2, b_cat2,
      W_final, b_final, b_edge)


def kernel(x, edge_index, W_edge, b_edge, W_node, b_node,
           W_cat1, b_cat1, W_cat2, b_cat2, W_final, b_final):
    S2 = _sc_segment_sum(W_edge, edge_index[0], edge_index[1])
    return _tc_tail(S2, x,
                    W_cat1, b_cat1.reshape(1, H),
                    W_node, b_node.reshape(1, H),
                    W_cat2, b_cat2.reshape(1, H),
                    W_final, b_final.reshape(1, OUT),
                    b_edge.reshape(1, H))
